# baseline (speedup 1.0000x reference)
import functools

import jax
import jax.numpy as jnp
from jax import lax
from jax.experimental import pallas as pl
from jax.experimental.pallas import tpu as pltpu
from jax.experimental.pallas import tpu_sc as plsc

B, L, V, D = 16384, 200, 8, 16

NC, NS = 2, 16
NW = NC * NS
BW = B // NW
LB = 8
NCHUNK = L // LB
GPL = BW // 16
NGRP = LB * GPL
VP = V // 2


def _sc_body(x2_hbm, e_hbm, w_hbm, bp_hbm, o_hbm,
             ltbuf, eb, wb, bb, xA, xB, oA, oB, sxA, sxB, soA, soB):
    c = lax.axis_index("c")
    s = lax.axis_index("s")
    wid = s * NC + c
    b0 = wid * BW

    def fire_x(ci, xbuf, sem):
        ci = jnp.minimum(ci, NCHUNK - 1)
        pltpu.async_copy(
            x2_hbm.at[pl.ds(ci * LB, LB), pl.ds(b0, BW)], xbuf, sem
        )

    fire_x(jnp.int32(0), xA, sxA)
    fire_x(jnp.int32(1), xB, sxB)

    pltpu.sync_copy(e_hbm, eb)
    pltpu.sync_copy(w_hbm, wb)
    pltpu.sync_copy(bp_hbm, bb)
    iota = lax.iota(jnp.int32, 16)
    k_idx = jnp.bitwise_and(iota, 7) * D
    hi = jnp.where(iota >= 8, 1, 0)
    for t in range(VP):
        v_idx = (2 * t + hi) * D
        acc = plsc.load_gather(bb, [2 * t + hi])
        for d in range(D):
            ed = plsc.load_gather(eb, [k_idx + d])
            wd = plsc.load_gather(wb, [v_idx + d])
            acc = acc + ed * wd
        ltbuf[pl.ds(16 * t, 16)] = acc

    def wait_x(xbuf, sem):
        pltpu.make_async_copy(
            x2_hbm.at[pl.ds(0, LB), pl.ds(b0, BW)], xbuf, sem
        ).wait()

    def wait_o(obuf, sem):
        pltpu.make_async_copy(
            obuf, o_hbm.at[pl.ds(0, LB), :, pl.ds(b0, BW)], sem
        ).wait()

    def compute(xbuf, obuf):
        @plsc.parallel_loop(0, NGRP, unroll=4)
        def grp(g):
            l = g // GPL
            bo = (g % GPL) * 16
            xv = xbuf[l, pl.ds(bo, 16)]
            for v in range(V):
                obuf[l, v, pl.ds(bo, 16)] = plsc.load_gather(
                    ltbuf, [xv + (V * v)]
                )

    def do_chunk(ci, xbuf, obuf, sx, so):
        wait_x(xbuf, sx)
        compute(xbuf, obuf)
        pltpu.async_copy(
            obuf, o_hbm.at[pl.ds(ci * LB, LB), :, pl.ds(b0, BW)], so
        )
        fire_x(ci + 2, xbuf, sx)

    do_chunk(jnp.int32(0), xA, oA, sxA, soA)
    do_chunk(jnp.int32(1), xB, oB, sxB, soB)

    def body(h, carry):
        wait_o(oA, soA)
        do_chunk(2 * h, xA, oA, sxA, soA)
        wait_o(oB, soB)
        do_chunk(2 * h + 1, xB, oB, sxB, soB)
        return carry

    lax.fori_loop(1, NCHUNK // 2, body, 0)

    if NCHUNK % 2:
        wait_o(oA, soA)
        do_chunk(jnp.int32(NCHUNK - 1), xA, oA, sxA, soA)

    wait_o(oA, soA)
    wait_o(oB, soB)
    wait_x(xA, sxA)
    wait_x(xB, sxB)


@functools.partial(jax.jit, static_argnames=())
def kernel(x, emb_table, W, b):
    ef = emb_table.reshape(V * D)
    wf = W.reshape(V * D)
    bp = jnp.pad(b, (0, V))
    x2 = x.astype(jnp.int32).T

    mesh = plsc.VectorSubcoreMesh(core_axis_name="c", subcore_axis_name="s")
    o3 = pl.kernel(
        _sc_body,
        out_type=jax.ShapeDtypeStruct((L, V, B), jnp.float32),
        mesh=mesh,
        compiler_params=pltpu.CompilerParams(needs_layout_passes=False),
        scratch_types=[
            pltpu.VMEM((V * V,), jnp.float32),
            pltpu.VMEM((V * D,), jnp.float32),
            pltpu.VMEM((V * D,), jnp.float32),
            pltpu.VMEM((16,), jnp.float32),
            pltpu.VMEM((LB, BW), jnp.int32),
            pltpu.VMEM((LB, BW), jnp.int32),
            pltpu.VMEM((LB, V, BW), jnp.float32),
            pltpu.VMEM((LB, V, BW), jnp.float32),
            pltpu.SemaphoreType.DMA,
            pltpu.SemaphoreType.DMA,
            pltpu.SemaphoreType.DMA,
            pltpu.SemaphoreType.DMA,
        ],
    )(x2, ef, wf, bp)
    return jnp.transpose(o3, (2, 0, 1))


# speedup vs baseline: 1.0031x; 1.0031x over previous
import functools

import jax
import jax.numpy as jnp
from jax import lax
from jax.experimental import pallas as pl
from jax.experimental.pallas import tpu as pltpu
from jax.experimental.pallas import tpu_sc as plsc

B, L, V, D = 16384, 200, 8, 16

NC, NS = 2, 16
NW = NC * NS
BW = B // NW
LB = 8
NCHUNK = L // LB
GPL = BW // 16
NGRP = LB * GPL
VP = V // 2


def _sc_body(x2_hbm, e_hbm, w_hbm, bp_hbm, o_hbm,
             ltbuf, eb, wb, bb, xA, xB, oA, oB, sxA, sxB, soA, soB):
    c = lax.axis_index("c")
    s = lax.axis_index("s")
    wid = s * NC + c
    b0 = wid * BW

    def fire_x(ci, xbuf, sem):
        ci = jnp.minimum(ci, NCHUNK - 1)
        pltpu.async_copy(
            x2_hbm.at[pl.ds(ci * LB, LB), pl.ds(b0, BW)], xbuf, sem
        )

    fire_x(jnp.int32(0), xA, sxA)
    fire_x(jnp.int32(1), xB, sxB)

    pltpu.sync_copy(e_hbm, eb)
    pltpu.sync_copy(w_hbm, wb)
    pltpu.sync_copy(bp_hbm, bb)
    iota = lax.iota(jnp.int32, 16)
    k_idx = jnp.bitwise_and(iota, 7) * D
    hi = jnp.where(iota >= 8, 1, 0)
    for t in range(VP):
        v_idx = (2 * t + hi) * D
        acc = plsc.load_gather(bb, [2 * t + hi])
        for d in range(D):
            ed = plsc.load_gather(eb, [k_idx + d])
            wd = plsc.load_gather(wb, [v_idx + d])
            acc = acc + ed * wd
        ltbuf[pl.ds(16 * t, 16)] = acc

    def wait_x(xbuf, sem):
        pltpu.make_async_copy(
            x2_hbm.at[pl.ds(0, LB), pl.ds(b0, BW)], xbuf, sem
        ).wait()

    def wait_o(obuf, sem):
        pltpu.make_async_copy(
            obuf, o_hbm.at[pl.ds(0, LB), :, pl.ds(b0, BW)], sem
        ).wait()

    def compute(xbuf, obuf):
        @plsc.parallel_loop(0, NGRP, unroll=8)
        def grp(g):
            l = g // GPL
            bo = (g % GPL) * 16
            xv = xbuf[l, pl.ds(bo, 16)]
            for v in range(V):
                obuf[l, v, pl.ds(bo, 16)] = plsc.load_gather(
                    ltbuf, [xv + (V * v)]
                )

    def do_chunk(ci, xbuf, obuf, sx, so):
        wait_x(xbuf, sx)
        compute(xbuf, obuf)
        pltpu.async_copy(
            obuf, o_hbm.at[pl.ds(ci * LB, LB), :, pl.ds(b0, BW)], so
        )
        fire_x(ci + 2, xbuf, sx)

    do_chunk(jnp.int32(0), xA, oA, sxA, soA)
    do_chunk(jnp.int32(1), xB, oB, sxB, soB)

    def body(h, carry):
        wait_o(oA, soA)
        do_chunk(2 * h, xA, oA, sxA, soA)
        wait_o(oB, soB)
        do_chunk(2 * h + 1, xB, oB, sxB, soB)
        return carry

    lax.fori_loop(1, NCHUNK // 2, body, 0)

    if NCHUNK % 2:
        wait_o(oA, soA)
        do_chunk(jnp.int32(NCHUNK - 1), xA, oA, sxA, soA)

    wait_o(oA, soA)
    wait_o(oB, soB)
    wait_x(xA, sxA)
    wait_x(xB, sxB)


@functools.partial(jax.jit, static_argnames=())
def kernel(x, emb_table, W, b):
    ef = emb_table.reshape(V * D)
    wf = W.reshape(V * D)
    bp = jnp.pad(b, (0, V))
    x2 = x.astype(jnp.int32).T

    mesh = plsc.VectorSubcoreMesh(core_axis_name="c", subcore_axis_name="s")
    o3 = pl.kernel(
        _sc_body,
        out_type=jax.ShapeDtypeStruct((L, V, B), jnp.float32),
        mesh=mesh,
        compiler_params=pltpu.CompilerParams(needs_layout_passes=False),
        scratch_types=[
            pltpu.VMEM((V * V,), jnp.float32),
            pltpu.VMEM((V * D,), jnp.float32),
            pltpu.VMEM((V * D,), jnp.float32),
            pltpu.VMEM((16,), jnp.float32),
            pltpu.VMEM((LB, BW), jnp.int32),
            pltpu.VMEM((LB, BW), jnp.int32),
            pltpu.VMEM((LB, V, BW), jnp.float32),
            pltpu.VMEM((LB, V, BW), jnp.float32),
            pltpu.SemaphoreType.DMA,
            pltpu.SemaphoreType.DMA,
            pltpu.SemaphoreType.DMA,
            pltpu.SemaphoreType.DMA,
        ],
    )(x2, ef, wf, bp)
    return jnp.transpose(o3, (2, 0, 1))


# speedup vs baseline: 1.0106x; 1.0075x over previous
import functools

import jax
import jax.numpy as jnp
from jax import lax
from jax.experimental import pallas as pl
from jax.experimental.pallas import tpu as pltpu
from jax.experimental.pallas import tpu_sc as plsc

B, L, V, D = 16384, 200, 8, 16

NC, NS = 2, 16
NW = NC * NS
BW = B // NW
LB = 8
NCHUNK = L // LB
GPL = BW // 16
NGRP = LB * GPL


def _lt_body(emb_ref, wt_ref, b_ref, o_ref):
    o_ref[...] = (
        jnp.dot(emb_ref[...], wt_ref[...], preferred_element_type=jnp.float32)
        + b_ref[...]
    )


def _fused_table(emb_table, W, b):
    return pl.pallas_call(
        _lt_body,
        out_shape=jax.ShapeDtypeStruct((V, V), jnp.float32),
    )(emb_table, W.T, b.reshape(1, V))


def _sc_body(x2_hbm, t_hbm, o_hbm, ltbuf, xA, xB, oA, oB, sxA, sxB, soA, soB):
    c = lax.axis_index("c")
    s = lax.axis_index("s")
    wid = s * NC + c
    b0 = wid * BW

    def fire_x(ci, xbuf, sem):
        ci = jnp.minimum(ci, NCHUNK - 1)
        pltpu.async_copy(
            x2_hbm.at[pl.ds(ci * LB, LB), pl.ds(b0, BW)], xbuf, sem
        )

    fire_x(jnp.int32(0), xA, sxA)
    fire_x(jnp.int32(1), xB, sxB)
    pltpu.sync_copy(t_hbm, ltbuf)

    def wait_x(xbuf, sem):
        pltpu.make_async_copy(
            x2_hbm.at[pl.ds(0, LB), pl.ds(b0, BW)], xbuf, sem
        ).wait()

    def wait_o(obuf, sem):
        pltpu.make_async_copy(
            obuf, o_hbm.at[pl.ds(0, LB), :, pl.ds(b0, BW)], sem
        ).wait()

    def compute(xbuf, obuf):
        @plsc.parallel_loop(0, NGRP, unroll=8)
        def grp(g):
            l = g // GPL
            bo = (g % GPL) * 16
            xv = xbuf[l, pl.ds(bo, 16)]
            for v in range(V):
                obuf[l, v, pl.ds(bo, 16)] = plsc.load_gather(
                    ltbuf, [xv + (V * v)]
                )

    def do_chunk(ci, xbuf, obuf, sx, so):
        wait_x(xbuf, sx)
        compute(xbuf, obuf)
        pltpu.async_copy(
            obuf, o_hbm.at[pl.ds(ci * LB, LB), :, pl.ds(b0, BW)], so
        )
        fire_x(ci + 2, xbuf, sx)

    do_chunk(jnp.int32(0), xA, oA, sxA, soA)
    do_chunk(jnp.int32(1), xB, oB, sxB, soB)

    def body(h, carry):
        wait_o(oA, soA)
        do_chunk(2 * h, xA, oA, sxA, soA)
        wait_o(oB, soB)
        do_chunk(2 * h + 1, xB, oB, sxB, soB)
        return carry

    lax.fori_loop(1, NCHUNK // 2, body, 0)

    if NCHUNK % 2:
        wait_o(oA, soA)
        do_chunk(jnp.int32(NCHUNK - 1), xA, oA, sxA, soA)

    wait_o(oA, soA)
    wait_o(oB, soB)
    wait_x(xA, sxA)
    wait_x(xB, sxB)


@functools.partial(jax.jit, static_argnames=())
def kernel(x, emb_table, W, b):
    lt = _fused_table(emb_table, W, b)
    tflat = jnp.transpose(lt).reshape(V * V)
    x2 = x.astype(jnp.int32).T

    mesh = plsc.VectorSubcoreMesh(core_axis_name="c", subcore_axis_name="s")
    o3 = pl.kernel(
        _sc_body,
        out_type=jax.ShapeDtypeStruct((L, V, B), jnp.float32),
        mesh=mesh,
        compiler_params=pltpu.CompilerParams(needs_layout_passes=False),
        scratch_types=[
            pltpu.VMEM((V * V,), jnp.float32),
            pltpu.VMEM((LB, BW), jnp.int32),
            pltpu.VMEM((LB, BW), jnp.int32),
            pltpu.VMEM((LB, V, BW), jnp.float32),
            pltpu.VMEM((LB, V, BW), jnp.float32),
            pltpu.SemaphoreType.DMA,
            pltpu.SemaphoreType.DMA,
            pltpu.SemaphoreType.DMA,
            pltpu.SemaphoreType.DMA,
        ],
    )(x2, tflat)
    return jnp.transpose(o3, (2, 0, 1))
